# baseline (speedup 1.0000x reference)
import jax
import jax.numpy as jnp
from jax.experimental import pallas as pl
from jax.experimental.pallas import tpu as pltpu

_B = 8
_C = 192
_G = 96
_HW = 224 * 224


def _copy_body(idx_ref, x1_ref, x2_ref, o1_ref, o2_ref):
    o1_ref[...] = x1_ref[...]
    o2_ref[...] = x2_ref[...]


def kernel(x, fp_index1, fp_index2):
    b, c, h, w = x.shape
    g = fp_index1.shape[0]
    hw = h * w
    sub = hw // 128
    xr = x.reshape(b * c, sub, 128)
    idx = jnp.concatenate(
        [fp_index1.astype(jnp.int32), fp_index2.astype(jnp.int32)]
    )

    blk = (1, sub, 128)
    grid_spec = pltpu.PrefetchScalarGridSpec(
        num_scalar_prefetch=1,
        grid=(b * g,),
        in_specs=[
            pl.BlockSpec(blk, lambda i, idx: (i // g * c + idx[i % g], 0, 0)),
            pl.BlockSpec(blk, lambda i, idx: (i // g * c + idx[g + i % g], 0, 0)),
        ],
        out_specs=[
            pl.BlockSpec(blk, lambda i, idx: (i, 0, 0)),
            pl.BlockSpec(blk, lambda i, idx: (i, 0, 0)),
        ],
    )

    out1, out2 = pl.pallas_call(
        _copy_body,
        grid_spec=grid_spec,
        out_shape=[
            jax.ShapeDtypeStruct((b * g, sub, 128), x.dtype),
            jax.ShapeDtypeStruct((b * g, sub, 128), x.dtype),
        ],
    )(idx, xr, xr)

    return out1.reshape(b, g, h, w), out2.reshape(b, g, h, w)


# speedup vs baseline: 1.2802x; 1.2802x over previous
import jax
import jax.numpy as jnp
from jax.experimental import pallas as pl
from jax.experimental.pallas import tpu as pltpu

_R = 4


def _copy_body(idx_ref, *refs):
    n = _R
    x1 = refs[:n]
    x2 = refs[n : 2 * n]
    o1 = refs[2 * n]
    o2 = refs[2 * n + 1]
    for r in range(n):
        o1[r, :, :] = x1[r][0]
        o2[r, :, :] = x2[r][0]


def kernel(x, fp_index1, fp_index2):
    b, c, h, w = x.shape
    g = fp_index1.shape[0]
    hw = h * w
    sub = hw // 128
    xr = x.reshape(b * c, sub, 128)
    idx = jnp.concatenate(
        [fp_index1.astype(jnp.int32), fp_index2.astype(jnp.int32)]
    )

    blk = (1, sub, 128)

    def src_map(r, off):
        def f(i, idx):
            n = i * _R + r
            return (n // g * c + idx[off + n % g], 0, 0)

        return f

    in_specs = [pl.BlockSpec(blk, src_map(r, 0)) for r in range(_R)]
    in_specs += [pl.BlockSpec(blk, src_map(r, g)) for r in range(_R)]

    grid_spec = pltpu.PrefetchScalarGridSpec(
        num_scalar_prefetch=1,
        grid=(b * g // _R,),
        in_specs=in_specs,
        out_specs=[
            pl.BlockSpec((_R, sub, 128), lambda i, idx: (i, 0, 0)),
            pl.BlockSpec((_R, sub, 128), lambda i, idx: (i, 0, 0)),
        ],
    )

    out1, out2 = pl.pallas_call(
        _copy_body,
        grid_spec=grid_spec,
        out_shape=[
            jax.ShapeDtypeStruct((b * g, sub, 128), x.dtype),
            jax.ShapeDtypeStruct((b * g, sub, 128), x.dtype),
        ],
    )(idx, *([xr] * (2 * _R)))

    return out1.reshape(b, g, h, w), out2.reshape(b, g, h, w)
